# baseline (speedup 1.0000x reference)
import jax
import jax.numpy as jnp
from jax.experimental import pallas as pl
from jax.experimental.pallas import tpu as pltpu


def _gnn_kernel(x_hbm, adj_ref, wj_ref, w1_ref, w2_ref, out_ref,
                acc_ref, xs_ref, y_ref, sem):
    b = pl.program_id(0)
    nb = pl.num_programs(0)
    nk = adj_ref.shape[1]
    slot = jax.lax.rem(b, 2)
    nxt = jax.lax.rem(b + 1, 2)

    nseg = sem.shape[2]
    seg = x_hbm.shape[1] // nseg

    def _issue(bb, ss):
        for k in range(nk):
            for s in range(nseg):
                pltpu.make_async_copy(
                    x_hbm.at[bb, pl.ds(s * seg, seg), k, :],
                    xs_ref.at[ss, k, pl.ds(s * seg, seg), :],
                    sem.at[ss, k, s],
                ).start()

    @pl.when(b == 0)
    def _():
        _issue(0, 0)

    @pl.when(b + 1 < nb)
    def _():
        _issue(b + 1, nxt)

    wj = wj_ref[0]

    for k in range(nk):
        for s in range(nseg):
            pltpu.make_async_copy(
                x_hbm.at[b, pl.ds(s * seg, seg), k, :],
                xs_ref.at[slot, k, pl.ds(s * seg, seg), :],
                sem.at[slot, k, s],
            ).wait()
        xs = xs_ref[slot, k].astype(jnp.bfloat16)
        y_ref[k] = jnp.dot(
            xs, w1_ref[...], preferred_element_type=jnp.float32
        ).astype(jnp.bfloat16)

    for k in range(nk):
        a = adj_ref[0, k].astype(jnp.bfloat16)
        m = jnp.dot(a, y_ref[k], preferred_element_type=jnp.float32)
        phi = jnp.maximum(m, 0.0) * wj[:, k:k + 1].astype(jnp.float32)
        if k < 2:
            acc_ref[k] = phi
        else:
            acc_ref[k % 2] = acc_ref[k % 2] + phi

    s = (acc_ref[0] + acc_ref[1]).astype(jnp.bfloat16)
    out_ref[0] = jnp.dot(s, w2_ref[...], preferred_element_type=jnp.float32)


def kernel(x, adj, w_j, W1, W2):
    B, N, K, D = x.shape
    H = W1.shape[1]
    DOUT = W2.shape[1]

    adj_i8 = adj.view(jnp.int8)
    w1b = W1.astype(jnp.bfloat16)
    w2b = W2.astype(jnp.bfloat16)

    return pl.pallas_call(
        _gnn_kernel,
        grid=(B,),
        in_specs=[
            pl.BlockSpec(memory_space=pltpu.MemorySpace.HBM),
            pl.BlockSpec((1, K, N, N), lambda b: (b, 0, 0, 0)),
            pl.BlockSpec((1, N, K), lambda b: (b, 0, 0)),
            pl.BlockSpec((D, H), lambda b: (0, 0)),
            pl.BlockSpec((H, DOUT), lambda b: (0, 0)),
        ],
        out_specs=pl.BlockSpec((1, N, DOUT), lambda b: (b, 0, 0)),
        out_shape=jax.ShapeDtypeStruct((B, N, DOUT), jnp.float32),
        scratch_shapes=[
            pltpu.VMEM((2, N, H), jnp.float32),
            pltpu.VMEM((2, K, N, D), jnp.float32),
            pltpu.VMEM((K, N, H), jnp.bfloat16),
            pltpu.SemaphoreType.DMA((2, K, 4)),
        ],
    )(x, adj_i8, w_j, w1b, w2b)


# speedup vs baseline: 1.0144x; 1.0144x over previous
import jax
import jax.numpy as jnp
from jax.experimental import pallas as pl
from jax.experimental.pallas import tpu as pltpu


def _gnn_kernel(x_hbm, adj_ref, wj_ref, w1_ref, w2_ref, out_ref,
                acc_ref, xs_ref, y_ref, sem):
    b = pl.program_id(0)
    nb = pl.num_programs(0)
    nk = adj_ref.shape[1]
    slot = jax.lax.rem(b, 2)
    nxt = jax.lax.rem(b + 1, 2)

    def _issue(bb, ss):
        for k in range(nk):
            pltpu.make_async_copy(
                x_hbm.at[bb, :, k, :], xs_ref.at[ss, k], sem.at[ss, k]
            ).start()

    @pl.when(b == 0)
    def _():
        _issue(0, 0)

    @pl.when(b + 1 < nb)
    def _():
        _issue(b + 1, nxt)

    wj = wj_ref[0]

    for k in range(nk):
        pltpu.make_async_copy(
            x_hbm.at[b, :, k, :], xs_ref.at[slot, k], sem.at[slot, k]
        ).wait()
        xs = xs_ref[slot, k].astype(jnp.bfloat16)
        y_ref[k] = jnp.dot(
            xs, w1_ref[...], preferred_element_type=jnp.float32
        ).astype(jnp.bfloat16)

    for k in range(nk):
        a = adj_ref[0, k].astype(jnp.bfloat16)
        m = jnp.dot(a, y_ref[k], preferred_element_type=jnp.float32)
        phi = jnp.maximum(m, 0.0) * wj[:, k:k + 1].astype(jnp.float32)
        if k < 2:
            acc_ref[k] = phi
        else:
            acc_ref[k % 2] = acc_ref[k % 2] + phi

    s = (acc_ref[0] + acc_ref[1]).astype(jnp.bfloat16)
    out_ref[0] = jnp.dot(s, w2_ref[...], preferred_element_type=jnp.float32)


def kernel(x, adj, w_j, W1, W2):
    B, N, K, D = x.shape
    H = W1.shape[1]
    DOUT = W2.shape[1]

    adj_i8 = adj.view(jnp.int8)
    w1b = W1.astype(jnp.bfloat16)
    w2b = W2.astype(jnp.bfloat16)

    return pl.pallas_call(
        _gnn_kernel,
        grid=(B,),
        in_specs=[
            pl.BlockSpec(memory_space=pltpu.MemorySpace.HBM),
            pl.BlockSpec((1, K, N, N), lambda b: (b, 0, 0, 0)),
            pl.BlockSpec((1, N, K), lambda b: (b, 0, 0)),
            pl.BlockSpec((D, H), lambda b: (0, 0)),
            pl.BlockSpec((H, DOUT), lambda b: (0, 0)),
        ],
        out_specs=pl.BlockSpec((1, N, DOUT), lambda b: (b, 0, 0)),
        out_shape=jax.ShapeDtypeStruct((B, N, DOUT), jnp.float32),
        scratch_shapes=[
            pltpu.VMEM((2, N, H), jnp.float32),
            pltpu.VMEM((2, K, N, D), jnp.float32),
            pltpu.VMEM((K, N, H), jnp.bfloat16),
            pltpu.SemaphoreType.DMA((2, K)),
        ],
    )(x, adj_i8, w_j, w1b, w2b)


# speedup vs baseline: 1.0287x; 1.0141x over previous
import jax
import jax.numpy as jnp
from jax.experimental import pallas as pl
from jax.experimental.pallas import tpu as pltpu


def _gnn_kernel(x_hbm, adj_hbm, wj_ref, w1_ref, w2_ref, out_ref,
                acc_ref, xs_ref, adj_ref, y_ref, sem, sem_adj):
    b = pl.program_id(0)
    nb = pl.num_programs(0)
    nk = adj_hbm.shape[1]
    slot = jax.lax.rem(b, 2)
    nxt = jax.lax.rem(b + 1, 2)

    def _issue(bb, ss):
        for k in range(nk):
            pltpu.make_async_copy(
                x_hbm.at[bb, :, k, :], xs_ref.at[ss, k], sem.at[ss, k]
            ).start()
        pltpu.make_async_copy(
            adj_hbm.at[bb], adj_ref.at[ss], sem_adj.at[ss]
        ).start()

    @pl.when(b == 0)
    def _():
        _issue(0, 0)

    @pl.when(b + 1 < nb)
    def _():
        _issue(b + 1, nxt)

    wj = wj_ref[0]

    for k in range(nk):
        pltpu.make_async_copy(
            x_hbm.at[b, :, k, :], xs_ref.at[slot, k], sem.at[slot, k]
        ).wait()
        xs = xs_ref[slot, k].astype(jnp.bfloat16)
        y_ref[k] = jnp.dot(
            xs, w1_ref[...], preferred_element_type=jnp.float32
        ).astype(jnp.bfloat16)

    pltpu.make_async_copy(
        adj_hbm.at[b], adj_ref.at[slot], sem_adj.at[slot]
    ).wait()
    for k in range(nk):
        a = adj_ref[slot, k].astype(jnp.bfloat16)
        m = jnp.dot(a, y_ref[k], preferred_element_type=jnp.float32)
        phi = jnp.maximum(m, 0.0) * wj[:, k:k + 1].astype(jnp.float32)
        if k < 2:
            acc_ref[k] = phi
        else:
            acc_ref[k % 2] = acc_ref[k % 2] + phi

    s = (acc_ref[0] + acc_ref[1]).astype(jnp.bfloat16)
    out_ref[0] = jnp.dot(s, w2_ref[...], preferred_element_type=jnp.float32)


def kernel(x, adj, w_j, W1, W2):
    B, N, K, D = x.shape
    H = W1.shape[1]
    DOUT = W2.shape[1]

    adj_i8 = adj.view(jnp.int8)
    w1b = W1.astype(jnp.bfloat16)
    w2b = W2.astype(jnp.bfloat16)

    return pl.pallas_call(
        _gnn_kernel,
        grid=(B,),
        in_specs=[
            pl.BlockSpec(memory_space=pltpu.MemorySpace.HBM),
            pl.BlockSpec(memory_space=pltpu.MemorySpace.HBM),
            pl.BlockSpec((1, N, K), lambda b: (b, 0, 0)),
            pl.BlockSpec((D, H), lambda b: (0, 0)),
            pl.BlockSpec((H, DOUT), lambda b: (0, 0)),
        ],
        out_specs=pl.BlockSpec((1, N, DOUT), lambda b: (b, 0, 0)),
        out_shape=jax.ShapeDtypeStruct((B, N, DOUT), jnp.float32),
        scratch_shapes=[
            pltpu.VMEM((2, N, H), jnp.float32),
            pltpu.VMEM((2, K, N, D), jnp.float32),
            pltpu.VMEM((2, K, N, N), jnp.int8),
            pltpu.VMEM((K, N, H), jnp.bfloat16),
            pltpu.SemaphoreType.DMA((2, K)),
            pltpu.SemaphoreType.DMA((2,)),
        ],
    )(x, adj_i8, w_j, w1b, w2b)
